# baseline (speedup 1.0000x reference)
import functools

import jax
import jax.numpy as jnp
from jax import lax
from jax.experimental import pallas as pl
from jax.experimental.pallas import tpu as pltpu
from jax.experimental.pallas import tpu_sc as plsc

_B = 16384
_LANES = 16
_TILES = 16
_ROWS_PER_TILE = _B // _TILES
_STEPS = _ROWS_PER_TILE // _LANES


def _sc_body(ig_hbm, sc_hbm, sib_hbm,
             igm_hbm, scm_hbm, cnt_hbm, opn_hbm,
             ig_v, sc_v, igm_v, scm_v, sib_v, cnt_v, opn_v,
             tot_smem):
    cid = lax.axis_index("c")
    sid = lax.axis_index("s")

    @pl.when(cid == 0)
    def _core0():
        base = sid * _ROWS_PER_TILE
        pltpu.sync_copy(ig_hbm.at[pl.ds(2 * base, 2 * _ROWS_PER_TILE)], ig_v)
        pltpu.sync_copy(sc_hbm.at[pl.ds(2 * base, 2 * _ROWS_PER_TILE)], sc_v)
        pltpu.sync_copy(sib_hbm, sib_v)
        sib16 = sib_v[...]
        iota2 = 2 * lax.iota(jnp.int32, 16)

        def step(j, acc):
            idx = j * (2 * _LANES) + iota2 + sib16
            igx = plsc.load_gather(ig_v, [idx])
            scx = plsc.load_gather(sc_v, [idx])
            igm_v[pl.ds(j * _LANES, _LANES)] = igx
            scm_v[pl.ds(j * _LANES, _LANES)] = scx
            return acc + scx

        acc = lax.fori_loop(0, _STEPS, step, jnp.zeros((_LANES,), jnp.int32))

        pltpu.sync_copy(igm_v, igm_hbm.at[pl.ds(base, _ROWS_PER_TILE)])
        pltpu.sync_copy(scm_v, scm_hbm.at[pl.ds(base, _ROWS_PER_TILE)])

        my_sum = jnp.sum(acc)

        @pl.when(sid == 0)
        def _init():
            tot_smem[0] = jnp.int32(0)

        plsc.subcore_barrier()
        plsc.fetch_and_add(tot_smem.at[0], my_sum, subcore_id=0)
        plsc.subcore_barrier()

        @pl.when(sid == 0)
        def _finalize():
            total = tot_smem[0].astype(jnp.float32)
            cnt_v[...] = jnp.broadcast_to(total, (_LANES,))
            opn_v[...] = jnp.broadcast_to(
                (total > 0.0).astype(jnp.int32), (_LANES,))
            pltpu.sync_copy(cnt_v, cnt_hbm)
            pltpu.sync_copy(opn_v, opn_hbm)


@jax.jit
def _sc_call(parent_ig_matrix, parent_sc_matrix, sib16):
    mesh = plsc.VectorSubcoreMesh(core_axis_name="c", subcore_axis_name="s")
    run = pl.kernel(
        _sc_body,
        out_type=[
            jax.ShapeDtypeStruct((_B,), jnp.int32),
            jax.ShapeDtypeStruct((_B,), jnp.int32),
            jax.ShapeDtypeStruct((_LANES,), jnp.float32),
            jax.ShapeDtypeStruct((_LANES,), jnp.int32),
        ],
        mesh=mesh,
        scratch_types=[
            pltpu.VMEM((2 * _ROWS_PER_TILE,), jnp.int32),
            pltpu.VMEM((2 * _ROWS_PER_TILE,), jnp.int32),
            pltpu.VMEM((_ROWS_PER_TILE,), jnp.int32),
            pltpu.VMEM((_ROWS_PER_TILE,), jnp.int32),
            pltpu.VMEM((_LANES,), jnp.int32),
            pltpu.VMEM((_LANES,), jnp.float32),
            pltpu.VMEM((_LANES,), jnp.int32),
            pltpu.SMEM((1,), jnp.int32),
        ],
        compiler_params=pltpu.CompilerParams(needs_layout_passes=False),
        name="cign_masking_sc",
    )
    return run(parent_ig_matrix.reshape(-1), parent_sc_matrix.reshape(-1),
               sib16)


def kernel(f_input, h_input, parent_ig_matrix, parent_sc_matrix, sibling_index):
    sib16 = jnp.full((_LANES,), sibling_index, dtype=jnp.int32)
    igm, scm, cnt, opn = _sc_call(parent_ig_matrix, parent_sc_matrix, sib16)
    sample_count = cnt[0]
    is_node_open = opn[0].astype(jnp.bool_)
    return (f_input, h_input, igm, scm, sample_count, is_node_open)
